# baseline (speedup 1.0000x reference)
import functools

import jax
import jax.numpy as jnp
from jax import lax
from jax.experimental import pallas as pl
from jax.experimental.pallas import tpu as pltpu
from jax.experimental.pallas import tpu_sc as plsc

B = 16384
DIM = 16
N_NEG = 5
NW = 32
BPW = B // NW
NCH = BPW // 128
NCH_NEG = BPW * N_NEG // 128

_mesh = plsc.VectorSubcoreMesh(core_axis_name="c", subcore_axis_name="s")


@functools.partial(
    pl.kernel,
    out_type=(
        jax.ShapeDtypeStruct((B, DIM), jnp.float32),
        jax.ShapeDtypeStruct((B, DIM), jnp.float32),
    ),
    mesh=_mesh,
    compiler_params=pltpu.CompilerParams(use_tc_tiling_on_sc=False),
    scratch_types=[
        pltpu.VMEM((NCH, 128), jnp.int32),
        pltpu.VMEM((NCH, 128), jnp.int32),
        pltpu.VMEM((NCH_NEG, 128), jnp.int32),
        pltpu.VMEM((BPW, DIM), jnp.float32),
        pltpu.VMEM((BPW, DIM), jnp.float32),
        pltpu.VMEM((BPW * N_NEG, DIM), jnp.float32),
        pltpu.VMEM((BPW, DIM), jnp.float32),
        pltpu.VMEM((BPW, DIM), jnp.float32),
        pltpu.SemaphoreType.DMA,
    ],
)
def _sc_gather(up_hbm, vp_hbm, vn_hbm, u_hbm, v_hbm, p_hbm, q_hbm,
               idx_u, idx_v, idx_n, urows, vrows, nrows, pbuf, qbuf, sem):
    wid = lax.axis_index("s") * 2 + lax.axis_index("c")
    base = wid * BPW

    pltpu.sync_copy(up_hbm.at[wid], idx_u)
    pltpu.sync_copy(vp_hbm.at[wid], idx_v)
    pltpu.sync_copy(vn_hbm.at[wid], idx_n)

    copies = []
    for j in range(NCH):
        copies.append(pltpu.async_copy(
            u_hbm.at[idx_u.at[j]], urows.at[pl.ds(j * 128, 128)], sem))
    for j in range(NCH):
        copies.append(pltpu.async_copy(
            v_hbm.at[idx_v.at[j]], vrows.at[pl.ds(j * 128, 128)], sem))
    for j in range(NCH_NEG):
        copies.append(pltpu.async_copy(
            v_hbm.at[idx_n.at[j]], nrows.at[pl.ds(j * 128, 128)], sem))
    for cp in copies:
        cp.wait()

    def body(i, _):
        u = urows[i, :]
        acc = nrows[5 * i, :] + nrows[5 * i + 1, :]
        acc = acc + nrows[5 * i + 2, :]
        acc = acc + nrows[5 * i + 3, :]
        acc = acc + nrows[5 * i + 4, :]
        pbuf[i, :] = u * vrows[i, :]
        qbuf[i, :] = u * acc
        return 0

    lax.fori_loop(0, BPW, body, 0)

    pltpu.sync_copy(pbuf, p_hbm.at[pl.ds(base, BPW)])
    pltpu.sync_copy(qbuf, q_hbm.at[pl.ds(base, BPW)])


def _tc_body(p_ref, q_ref, s_ref, o_ref):
    sel = s_ref[...]
    sc = jnp.dot(p_ref[...], sel, preferred_element_type=jnp.float32)
    ng = jnp.dot(q_ref[...], sel, preferred_element_type=jnp.float32)

    def logsig(x):
        return jnp.minimum(x, 0.0) - jnp.log1p(jnp.exp(-jnp.abs(x)))

    total = jnp.sum(logsig(sc)) + jnp.sum(logsig(-ng))
    o_ref[0, 0] = -total / B


def kernel(u_pos, v_pos, v_neg, batch_size, U, V, cluster_means):
    del batch_size, cluster_means
    up = u_pos.astype(jnp.int32).reshape(NW, NCH, 128)
    vp = v_pos.astype(jnp.int32).reshape(NW, NCH, 128)
    vn = v_neg.astype(jnp.int32).reshape(NW, NCH_NEG, 128)
    p, q = _sc_gather(up, vp, vn, U, V)

    sel = (lax.broadcasted_iota(jnp.int32, (128, 8), 0) // 16
           == lax.broadcasted_iota(jnp.int32, (128, 8), 1)).astype(jnp.float32)
    out = pl.pallas_call(
        _tc_body,
        out_shape=jax.ShapeDtypeStruct((1, 1), jnp.float32),
        out_specs=pl.BlockSpec(memory_space=pltpu.SMEM),
    )(p.reshape(B * DIM // 128, 128), q.reshape(B * DIM // 128, 128), sel)
    return out[0, 0]
